# baseline (speedup 1.0000x reference)
import jax
import jax.numpy as jnp
from jax.experimental import pallas as pl


def _add_body(in_ref, table_ref, out_ref):
    out_ref[...] = in_ref[...] + table_ref[...]


def kernel(inputs, table):
    B, P, D = inputs.shape
    return pl.pallas_call(
        _add_body,
        grid=(B,),
        in_specs=[
            pl.BlockSpec((1, P, D), lambda b: (b, 0, 0)),
            pl.BlockSpec((P, D), lambda b: (0, 0)),
        ],
        out_specs=pl.BlockSpec((1, P, D), lambda b: (b, 0, 0)),
        out_shape=jax.ShapeDtypeStruct((B, P, D), inputs.dtype),
    )(inputs, table)


# speedup vs baseline: 1.6197x; 1.6197x over previous
import jax
import jax.numpy as jnp
from jax.experimental import pallas as pl


def _add_body(in_ref, table_ref, out_ref):
    out_ref[...] = in_ref[...] + table_ref[...][None]


def kernel(inputs, table):
    B, P, D = inputs.shape
    BB = 4
    return pl.pallas_call(
        _add_body,
        grid=(B // BB,),
        in_specs=[
            pl.BlockSpec((BB, P, D), lambda b: (b, 0, 0)),
            pl.BlockSpec((P, D), lambda b: (0, 0)),
        ],
        out_specs=pl.BlockSpec((BB, P, D), lambda b: (b, 0, 0)),
        out_shape=jax.ShapeDtypeStruct((B, P, D), inputs.dtype),
    )(inputs, table)


# speedup vs baseline: 1.6947x; 1.0463x over previous
import jax
import jax.numpy as jnp
from jax.experimental import pallas as pl


def _add_body(in_ref, table_ref, out_ref):
    out_ref[...] = in_ref[...] + table_ref[...][None]


def kernel(inputs, table):
    B, P, D = inputs.shape
    BB = 8
    return pl.pallas_call(
        _add_body,
        grid=(B // BB,),
        in_specs=[
            pl.BlockSpec((BB, P, D), lambda b: (b, 0, 0)),
            pl.BlockSpec((P, D), lambda b: (0, 0)),
        ],
        out_specs=pl.BlockSpec((BB, P, D), lambda b: (b, 0, 0)),
        out_shape=jax.ShapeDtypeStruct((B, P, D), inputs.dtype),
    )(inputs, table)


# speedup vs baseline: 1.7617x; 1.0395x over previous
import jax
import jax.numpy as jnp
from jax.experimental import pallas as pl


def _add_body(in_ref, table_ref, out_ref):
    out_ref[...] = in_ref[...] + table_ref[...][None]


def kernel(inputs, table):
    B, P, D = inputs.shape
    BB = 16
    return pl.pallas_call(
        _add_body,
        grid=(B // BB,),
        in_specs=[
            pl.BlockSpec((BB, P, D), lambda b: (b, 0, 0)),
            pl.BlockSpec((P, D), lambda b: (0, 0)),
        ],
        out_specs=pl.BlockSpec((BB, P, D), lambda b: (b, 0, 0)),
        out_shape=jax.ShapeDtypeStruct((B, P, D), inputs.dtype),
    )(inputs, table)
